# baseline (speedup 1.0000x reference)
import functools

import jax
import jax.numpy as jnp
from jax import lax
from jax.experimental import pallas as pl
from jax.experimental.pallas import tpu as pltpu
from jax.experimental.pallas import tpu_sc as plsc

_NUM_EMB = 1024
_DIM = 64
_N_TOK = 32768
_HALF = _N_TOK // 2
_EPS = 1e-12

_BLK = 2048
_GRID = _HALF // _BLK
_CC = 256
_NCC = _NUM_EMB // _CC
_ACC_R = _CC
_ACC_C = _NCC


def _core(i, x_ref, c_ref, idx_ref, acc_ref, dist_ref, cnb_ref, acc_init):
    @pl.when(i == 0)
    def _precompute():
        for j in range(_NCC):
            cj = c_ref[pl.ds(j * _CC, _CC), :]
            cn = jnp.sum(cj * cj, axis=1, keepdims=True)
            cnb_ref[pl.ds(j * _CC, _CC), :] = jnp.broadcast_to(cn, (_CC, _BLK))
        acc_ref[...] = acc_init()

    x = x_ref[...]
    xm = -2.0 * x
    xsq = x * x
    ones_row = jnp.ones((1, _DIM), jnp.float32)
    x2row = lax.dot_general(
        ones_row, xsq, (((1,), (1,)), ((), ())), preferred_element_type=jnp.float32
    )

    run_min = jnp.full((1, _BLK), jnp.inf, jnp.float32)
    for j in range(_NCC):
        cj = c_ref[pl.ds(j * _CC, _CC), :]
        msim = lax.dot_general(
            cj, xm, (((1,), (1,)), ((), ())), preferred_element_type=jnp.float32
        )
        dist = (x2row + cnb_ref[pl.ds(j * _CC, _CC), :]) + msim
        dist_ref[pl.ds(j * _CC, _CC), :] = dist
        run_min = jnp.minimum(run_min, jnp.min(dist, axis=0, keepdims=True))

    ones = jnp.ones((_BLK, 1), jnp.float32)
    run_arg = jnp.full((1, _BLK), _NUM_EMB, jnp.int32)
    cnts = []
    for j in range(_NCC):
        dist = dist_ref[pl.ds(j * _CC, _CC), :]
        hit = dist == run_min
        row_iota = lax.broadcasted_iota(jnp.int32, (_CC, _BLK), 0)
        cand = jnp.where(hit, row_iota + j * _CC, _NUM_EMB)
        run_arg = jnp.minimum(run_arg, jnp.min(cand, axis=0, keepdims=True))
        eq = jnp.where(hit, 1.0, 0.0)
        cnts.append(
            lax.dot_general(
                eq, ones, (((1,), (0,)), ((), ())), preferred_element_type=jnp.float32
            )
        )
    idx_ref[...] = run_arg.reshape(_BLK)
    acc_ref[...] += jnp.concatenate(cnts, axis=1)


def _first_body(x_ref, c_ref, idx_ref, acc_out_ref, dist_ref, cnb_ref):
    i = pl.program_id(0)
    _core(
        i, x_ref, c_ref, idx_ref, acc_out_ref, dist_ref, cnb_ref,
        lambda: jnp.zeros((_ACC_R, _ACC_C), jnp.float32),
    )


def _second_body(
    x_ref, c_ref, used_ref, acc_in_ref, idx_ref, used_out_ref, perp_ref,
    acc_ref, dist_ref, cnb_ref,
):
    i = pl.program_id(0)
    _core(
        i, x_ref, c_ref, idx_ref, acc_ref, dist_ref, cnb_ref,
        lambda: acc_in_ref[...],
    )

    @pl.when(i == _GRID - 1)
    def _finish():
        counts = acc_ref[...]
        used_out_ref[...] = used_ref[...] + counts.astype(jnp.int32)
        p = counts * (1.0 / _N_TOK)
        perp = jnp.exp(-jnp.sum(p * jnp.log(p + _EPS), axis=(0, 1), keepdims=True))
        perp_ref[...] = perp


_X_SPEC = pl.BlockSpec((_BLK, _DIM), lambda i: (i, 0))
_C_SPEC = pl.BlockSpec((_NUM_EMB, _DIM), lambda i: (0, 0))
_ACC_SPEC = pl.BlockSpec((_ACC_R, _ACC_C), lambda i: (0, 0))
_IDX_SPEC = pl.BlockSpec((_BLK,), lambda i: (i,))
_SCRATCH = [
    pltpu.VMEM((_NUM_EMB, _BLK), jnp.float32),
    pltpu.VMEM((_NUM_EMB, _BLK), jnp.float32),
]


def _argmin_first(flat, codebooks):
    return pl.pallas_call(
        _first_body,
        grid=(_GRID,),
        in_specs=[_X_SPEC, _C_SPEC],
        out_specs=[_IDX_SPEC, _ACC_SPEC],
        out_shape=[
            jax.ShapeDtypeStruct((_HALF,), jnp.int32),
            jax.ShapeDtypeStruct((_ACC_R, _ACC_C), jnp.float32),
        ],
        scratch_shapes=_SCRATCH,
    )(flat, codebooks)


def _argmin_second(flat, codebooks, used_t, acc_prev):
    return pl.pallas_call(
        _second_body,
        grid=(_GRID,),
        in_specs=[_X_SPEC, _C_SPEC, _ACC_SPEC, _ACC_SPEC],
        out_specs=[_IDX_SPEC, _ACC_SPEC, pl.BlockSpec((1, 1), lambda i: (0, 0))],
        out_shape=[
            jax.ShapeDtypeStruct((_HALF,), jnp.int32),
            jax.ShapeDtypeStruct((_ACC_R, _ACC_C), jnp.int32),
            jax.ShapeDtypeStruct((1, 1), jnp.float32),
        ],
        scratch_shapes=[pltpu.VMEM((_ACC_R, _ACC_C), jnp.float32)] + _SCRATCH,
    )(flat, codebooks, used_t, acc_prev)


_NW = 32
_BPW = _HALF // _NW
_CH = 128
_NCH = _BPW // _CH


@functools.lru_cache(maxsize=1)
def _get_sc_gather():
    info = plsc.get_sparse_core_info()
    nc = info.num_cores
    assert nc * info.num_subcores == _NW

    @functools.partial(
        pl.kernel,
        mesh=plsc.VectorSubcoreMesh(core_axis_name="c", subcore_axis_name="s"),
        out_type=jax.ShapeDtypeStruct((_HALF, _DIM), jnp.float32),
        scratch_types=[
            pltpu.VMEM((_BPW,), jnp.int32),
            pltpu.VMEM((_BPW, _DIM), jnp.float32),
            pltpu.SemaphoreType.DMA,
        ],
        compiler_params=pltpu.CompilerParams(use_tc_tiling_on_sc=False),
    )
    def _sc_gather(c_hbm, idx_hbm, out_hbm, idx_v, rows_v, sem):
        wid = lax.axis_index("s") * nc + lax.axis_index("c")
        base = wid * _BPW
        pltpu.sync_copy(idx_hbm.at[pl.ds(base, _BPW)], idx_v)
        handles = []
        for ch in range(_NCH):
            handles.append(
                pltpu.async_copy(
                    c_hbm.at[idx_v.at[pl.ds(ch * _CH, _CH)]],
                    rows_v.at[pl.ds(ch * _CH, _CH)],
                    sem,
                )
            )
        for h in handles:
            h.wait()
        pltpu.sync_copy(rows_v, out_hbm.at[pl.ds(base, _BPW)])

    return _sc_gather


def kernel(input_data, codebooks, codebooks_used):
    flat = input_data.reshape(-1, _DIM)
    used_t = codebooks_used.reshape(_NCC, _CC).T
    idx_a, acc_a = _argmin_first(flat[:_HALF], codebooks)
    quant_a = _get_sc_gather()(codebooks, idx_a)
    idx_b, used_out, perp = _argmin_second(flat[_HALF:], codebooks, used_t, acc_a)
    quant_b = _get_sc_gather()(codebooks, idx_b)
    quantized = jnp.concatenate([quant_a, quant_b], axis=0)
    quantized = quantized.reshape(input_data.shape[:-1] + (_DIM,))
    return (quantized, perp[0, 0], used_out.T.reshape(_NUM_EMB))


# speedup vs baseline: 1.1326x; 1.1326x over previous
import functools

import jax
import jax.numpy as jnp
from jax import lax
from jax.experimental import pallas as pl
from jax.experimental.pallas import tpu as pltpu
from jax.experimental.pallas import tpu_sc as plsc

_NUM_EMB = 1024
_DIM = 64
_N_TOK = 32768
_EPS = 1e-12

_BLK = 2048
_GRID = _N_TOK // _BLK
_CC = 256
_NCC = _NUM_EMB // _CC


def _argmin_body(
    x_ref, c_ref, used_ref, idx_ref, used_out_ref, perp_ref,
    acc_ref, dist_ref, cnb_ref,
):
    i = pl.program_id(0)

    @pl.when(i == 0)
    def _precompute():
        for j in range(_NCC):
            cj = c_ref[pl.ds(j * _CC, _CC), :]
            cn = jnp.sum(cj * cj, axis=1, keepdims=True)
            cnb_ref[pl.ds(j * _CC, _CC), :] = jnp.broadcast_to(cn, (_CC, _BLK))
        acc_ref[...] = jnp.zeros((_CC, _NCC), jnp.float32)

    x = x_ref[...]
    xm = -2.0 * x
    xsq = x * x
    ones_row = jnp.ones((1, _DIM), jnp.float32)
    x2row = lax.dot_general(
        ones_row, xsq, (((1,), (1,)), ((), ())), preferred_element_type=jnp.float32
    )

    run_min = jnp.full((1, _BLK), jnp.inf, jnp.float32)
    for j in range(_NCC):
        cj = c_ref[pl.ds(j * _CC, _CC), :]
        msim = lax.dot_general(
            cj, xm, (((1,), (1,)), ((), ())), preferred_element_type=jnp.float32
        )
        dist = (x2row + cnb_ref[pl.ds(j * _CC, _CC), :]) + msim
        dist_ref[pl.ds(j * _CC, _CC), :] = dist
        run_min = jnp.minimum(run_min, jnp.min(dist, axis=0, keepdims=True))

    ones = jnp.ones((_BLK, 1), jnp.float32)
    run_arg = jnp.full((1, _BLK), _NUM_EMB, jnp.int32)
    cnts = []
    for j in range(_NCC):
        dist = dist_ref[pl.ds(j * _CC, _CC), :]
        hit = dist == run_min
        row_iota = lax.broadcasted_iota(jnp.int32, (_CC, _BLK), 0)
        cand = jnp.where(hit, row_iota + j * _CC, _NUM_EMB)
        run_arg = jnp.minimum(run_arg, jnp.min(cand, axis=0, keepdims=True))
        eq = jnp.where(hit, 1.0, 0.0)
        cnts.append(
            lax.dot_general(
                eq, ones, (((1,), (0,)), ((), ())), preferred_element_type=jnp.float32
            )
        )
    idx_ref[...] = run_arg.reshape(_BLK)
    acc_ref[...] += jnp.concatenate(cnts, axis=1)

    @pl.when(i == _GRID - 1)
    def _finish():
        counts = acc_ref[...]
        used_out_ref[...] = used_ref[...] + counts.astype(jnp.int32)
        p = counts * (1.0 / _N_TOK)
        perp = jnp.exp(-jnp.sum(p * jnp.log(p + _EPS), axis=(0, 1), keepdims=True))
        perp_ref[...] = perp


def _argmin_counts(flat, codebooks, used_t):
    return pl.pallas_call(
        _argmin_body,
        grid=(_GRID,),
        in_specs=[
            pl.BlockSpec((_BLK, _DIM), lambda i: (i, 0)),
            pl.BlockSpec((_NUM_EMB, _DIM), lambda i: (0, 0)),
            pl.BlockSpec((_CC, _NCC), lambda i: (0, 0)),
        ],
        out_specs=[
            pl.BlockSpec((_BLK,), lambda i: (i,)),
            pl.BlockSpec((_CC, _NCC), lambda i: (0, 0)),
            pl.BlockSpec((1, 1), lambda i: (0, 0)),
        ],
        out_shape=[
            jax.ShapeDtypeStruct((_N_TOK,), jnp.int32),
            jax.ShapeDtypeStruct((_CC, _NCC), jnp.int32),
            jax.ShapeDtypeStruct((1, 1), jnp.float32),
        ],
        scratch_shapes=[
            pltpu.VMEM((_CC, _NCC), jnp.float32),
            pltpu.VMEM((_NUM_EMB, _BLK), jnp.float32),
            pltpu.VMEM((_NUM_EMB, _BLK), jnp.float32),
        ],
    )(flat, codebooks, used_t)


_NW = 32
_BPW = _N_TOK // _NW
_CH = 128
_NCH = _BPW // _CH


@functools.lru_cache(maxsize=1)
def _get_sc_gather():
    info = plsc.get_sparse_core_info()
    nc = info.num_cores
    assert nc * info.num_subcores == _NW

    @functools.partial(
        pl.kernel,
        mesh=plsc.VectorSubcoreMesh(core_axis_name="c", subcore_axis_name="s"),
        out_type=jax.ShapeDtypeStruct((_N_TOK, _DIM), jnp.float32),
        scratch_types=[
            pltpu.VMEM((_BPW,), jnp.int32),
            pltpu.VMEM((_BPW, _DIM), jnp.float32),
            pltpu.SemaphoreType.DMA,
        ],
        compiler_params=pltpu.CompilerParams(use_tc_tiling_on_sc=False),
    )
    def _sc_gather(c_hbm, idx_hbm, out_hbm, idx_v, rows_v, sem):
        wid = lax.axis_index("s") * nc + lax.axis_index("c")
        base = wid * _BPW
        pltpu.sync_copy(idx_hbm.at[pl.ds(base, _BPW)], idx_v)
        handles = []
        for ch in range(_NCH):
            handles.append(
                pltpu.async_copy(
                    c_hbm.at[idx_v.at[pl.ds(ch * _CH, _CH)]],
                    rows_v.at[pl.ds(ch * _CH, _CH)],
                    sem,
                )
            )
        for h in handles:
            h.wait()
        pltpu.sync_copy(rows_v, out_hbm.at[pl.ds(base, _BPW)])

    return _sc_gather


def kernel(input_data, codebooks, codebooks_used):
    flat = input_data.reshape(-1, _DIM)
    used_t = codebooks_used.reshape(_NCC, _CC).T
    idx_flat, used_out, perp = _argmin_counts(flat, codebooks, used_t)
    quantized = _get_sc_gather()(codebooks, idx_flat)
    quantized = quantized.reshape(input_data.shape[:-1] + (_DIM,))
    return (quantized, perp[0, 0], used_out.T.reshape(_NUM_EMB))
